# baseline (speedup 1.0000x reference)
import functools

import jax
import jax.numpy as jnp
from jax.experimental import pallas as pl
from jax.experimental.pallas import tpu as pltpu

_WIN = 5
_PAD = (_WIN - 1) // 2


def _vq_loss_body(x_ref, e_ref, out_ref, *, Tlen, P, tblk, nb, nsteps, scale):
    b = pl.program_id(0)
    ebf = e_ref[...]

    total = jnp.float32(0.0)
    acc_v = jnp.zeros((tblk,), jnp.float32)
    for i in range(nb):
        xbb = x_ref[i * P:(i + 1) * P, :]
        xb = xbb.astype(jnp.float32)

        s_all = jnp.sum(xb * xb)
        c0 = xb[:, 0:1]
        c1 = xb[:, 1:2]
        c2 = xb[:, Tlen - 2:Tlen - 1]
        c3 = xb[:, Tlen - 1:Tlen]
        corr = (2.0 * jnp.sum(c0 * c0) + jnp.sum(c1 * c1)
                + jnp.sum(c2 * c2) + 2.0 * jnp.sum(c3 * c3))
        total = total + 5.0 * s_all - corr

        z1 = jnp.zeros((P, 1), jnp.bfloat16)
        z2 = jnp.zeros((P, 2), jnp.bfloat16)
        ft = jnp.concatenate(
            [
                jnp.concatenate([z2, xbb[:, :Tlen - 2]], axis=1),
                jnp.concatenate([z1, xbb[:, :Tlen - 1]], axis=1),
                xbb,
                jnp.concatenate([xbb[:, 1:], z1], axis=1),
                jnp.concatenate([xbb[:, 2:], z2], axis=1),
                jnp.ones((1, Tlen), jnp.bfloat16),
            ],
            axis=0,
        )

        for t0 in range(0, Tlen, tblk):
            g = jax.lax.dot_general(
                ebf, ft[:, t0:t0 + tblk],
                dimension_numbers=(((0,), (0,)), ((), ())),
                preferred_element_type=jnp.float32,
            )
            acc_v = acc_v + jnp.max(g, axis=0)

    total = total - 2.0 * jnp.sum(acc_v)

    @pl.when(b == 0)
    def _init():
        out_ref[0, 0, 0] = 0.0

    out_ref[0, 0, 0] += total

    @pl.when(b == nsteps - 1)
    def _finish():
        out_ref[0, 0, 0] = out_ref[0, 0, 0] * scale


@jax.jit
def kernel(x, embedding):
    B, P, T = x.shape
    K, D = embedding.shape
    et = jnp.transpose(embedding.reshape(K, P, _WIN), (2, 1, 0)).reshape(D, K)
    e2 = jnp.sum(embedding * embedding, axis=1)
    ebf = jnp.concatenate([et, (-0.5 * e2)[None, :]], axis=0).astype(jnp.bfloat16)

    xf = x.astype(jnp.bfloat16).reshape(B * P, T)

    NB = 4
    nsteps = B // NB
    body = functools.partial(
        _vq_loss_body, Tlen=T, P=P, tblk=2048, nb=NB, nsteps=nsteps,
        scale=0.25 / (B * T * D),
    )
    out = pl.pallas_call(
        body,
        grid=(nsteps,),
        in_specs=[
            pl.BlockSpec((NB * P, T), lambda b: (b, 0)),
            pl.BlockSpec((D + 1, K), lambda b: (0, 0)),
        ],
        out_specs=pl.BlockSpec((1, 1, 1), lambda b: (0, 0, 0),
                               memory_space=pltpu.SMEM),
        out_shape=jax.ShapeDtypeStruct((1, 1, 1), jnp.float32),
    )(xf, ebf)
    return out.reshape(())


# speedup vs baseline: 1.0140x; 1.0140x over previous
import functools

import jax
import jax.numpy as jnp
from jax.experimental import pallas as pl
from jax.experimental.pallas import tpu as pltpu

_WIN = 5
_PAD = (_WIN - 1) // 2


def _vq_loss_body(x_ref, e_ref, out_ref, *, Tlen, P, tblk, nb, nsteps, scale):
    b = pl.program_id(0)
    ebf = e_ref[...]

    total = jnp.float32(0.0)
    acc_v = jnp.zeros((tblk,), jnp.float32)
    for i in range(nb):
        xbb = x_ref[i * P:(i + 1) * P, :]
        xb = xbb.astype(jnp.float32)

        s_all = jnp.sum(xb * xb)
        c0 = xb[:, 0:1]
        c1 = xb[:, 1:2]
        c2 = xb[:, Tlen - 2:Tlen - 1]
        c3 = xb[:, Tlen - 1:Tlen]
        corr = (2.0 * jnp.sum(c0 * c0) + jnp.sum(c1 * c1)
                + jnp.sum(c2 * c2) + 2.0 * jnp.sum(c3 * c3))
        total = total + 5.0 * s_all - corr

        z1 = jnp.zeros((P, 1), jnp.bfloat16)
        z2 = jnp.zeros((P, 2), jnp.bfloat16)
        ft = jnp.concatenate(
            [
                jnp.concatenate([z2, xbb[:, :Tlen - 2]], axis=1),
                jnp.concatenate([z1, xbb[:, :Tlen - 1]], axis=1),
                xbb,
                jnp.concatenate([xbb[:, 1:], z1], axis=1),
                jnp.concatenate([xbb[:, 2:], z2], axis=1),
                jnp.ones((1, Tlen), jnp.bfloat16),
            ],
            axis=0,
        )

        for t0 in range(0, Tlen, tblk):
            g = jax.lax.dot_general(
                ebf, ft[:, t0:t0 + tblk],
                dimension_numbers=(((0,), (0,)), ((), ())),
                preferred_element_type=jnp.float32,
            )
            acc_v = acc_v + jnp.max(g, axis=0)

    total = total - 2.0 * jnp.sum(acc_v)

    @pl.when(b == 0)
    def _init():
        out_ref[0, 0, 0] = 0.0

    out_ref[0, 0, 0] += total

    @pl.when(b == nsteps - 1)
    def _finish():
        out_ref[0, 0, 0] = out_ref[0, 0, 0] * scale


@jax.jit
def kernel(x, embedding):
    B, P, T = x.shape
    K, D = embedding.shape
    et = jnp.transpose(embedding.reshape(K, P, _WIN), (2, 1, 0)).reshape(D, K)
    e2 = jnp.sum(embedding * embedding, axis=1)
    ebf = jnp.concatenate([et, (-0.5 * e2)[None, :]], axis=0).astype(jnp.bfloat16)

    xf = x.astype(jnp.bfloat16).reshape(B * P, T)

    NB = 16
    nsteps = B // NB
    body = functools.partial(
        _vq_loss_body, Tlen=T, P=P, tblk=2048, nb=NB, nsteps=nsteps,
        scale=0.25 / (B * T * D),
    )
    out = pl.pallas_call(
        body,
        grid=(nsteps,),
        in_specs=[
            pl.BlockSpec((NB * P, T), lambda b: (b, 0)),
            pl.BlockSpec((D + 1, K), lambda b: (0, 0)),
        ],
        out_specs=pl.BlockSpec((1, 1, 1), lambda b: (0, 0, 0),
                               memory_space=pltpu.SMEM),
        out_shape=jax.ShapeDtypeStruct((1, 1, 1), jnp.float32),
    )(xf, ebf)
    return out.reshape(())


# speedup vs baseline: 1.0176x; 1.0035x over previous
import functools

import jax
import jax.numpy as jnp
from jax.experimental import pallas as pl
from jax.experimental.pallas import tpu as pltpu

_WIN = 5
_PAD = (_WIN - 1) // 2


def _vq_loss_body(x_ref, e_ref, out_ref, *, Tlen, P, tblk, nb, nsteps, scale):
    b = pl.program_id(0)
    ebf = e_ref[...]

    total = jnp.float32(0.0)
    acc_v = jnp.zeros((tblk,), jnp.float32)
    for i in range(nb):
        xbb = x_ref[i * P:(i + 1) * P, :]
        xb = xbb.astype(jnp.float32)

        s_all = jnp.sum(xb * xb)
        c0 = xb[:, 0:1]
        c1 = xb[:, 1:2]
        c2 = xb[:, Tlen - 2:Tlen - 1]
        c3 = xb[:, Tlen - 1:Tlen]
        corr = (2.0 * jnp.sum(c0 * c0) + jnp.sum(c1 * c1)
                + jnp.sum(c2 * c2) + 2.0 * jnp.sum(c3 * c3))
        total = total + 5.0 * s_all - corr

        z1 = jnp.zeros((P, 1), jnp.bfloat16)
        z2 = jnp.zeros((P, 2), jnp.bfloat16)
        ft = jnp.concatenate(
            [
                jnp.concatenate([z2, xbb[:, :Tlen - 2]], axis=1),
                jnp.concatenate([z1, xbb[:, :Tlen - 1]], axis=1),
                xbb,
                jnp.concatenate([xbb[:, 1:], z1], axis=1),
                jnp.concatenate([xbb[:, 2:], z2], axis=1),
                jnp.ones((1, Tlen), jnp.bfloat16),
            ],
            axis=0,
        )

        for t0 in range(0, Tlen, tblk):
            g = jax.lax.dot_general(
                ebf, ft[:, t0:t0 + tblk],
                dimension_numbers=(((0,), (0,)), ((), ())),
                preferred_element_type=jnp.float32,
            )
            acc_v = acc_v + jnp.max(g, axis=0)

    total = total - 2.0 * jnp.sum(acc_v)

    @pl.when(b == 0)
    def _init():
        out_ref[0, 0, 0] = 0.0

    out_ref[0, 0, 0] += total

    @pl.when(b == nsteps - 1)
    def _finish():
        out_ref[0, 0, 0] = out_ref[0, 0, 0] * scale


@jax.jit
def kernel(x, embedding):
    B, P, T = x.shape
    K, D = embedding.shape
    et = jnp.transpose(embedding.reshape(K, P, _WIN), (2, 1, 0)).reshape(D, K)
    e2 = jnp.sum(embedding * embedding, axis=1)
    ebf = jnp.concatenate([et, (-0.5 * e2)[None, :]], axis=0).astype(jnp.bfloat16)

    xf = x.astype(jnp.bfloat16).reshape(B * P, T)

    NB = 8
    nsteps = B // NB
    body = functools.partial(
        _vq_loss_body, Tlen=T, P=P, tblk=2048, nb=NB, nsteps=nsteps,
        scale=0.25 / (B * T * D),
    )
    out = pl.pallas_call(
        body,
        grid=(nsteps,),
        in_specs=[
            pl.BlockSpec((NB * P, T), lambda b: (b, 0)),
            pl.BlockSpec((D + 1, K), lambda b: (0, 0)),
        ],
        out_specs=pl.BlockSpec((1, 1, 1), lambda b: (0, 0, 0),
                               memory_space=pltpu.SMEM),
        out_shape=jax.ShapeDtypeStruct((1, 1, 1), jnp.float32),
    )(xf, ebf)
    return out.reshape(())
